# Initial kernel scaffold; baseline (speedup 1.0000x reference)
import jax
import jax.numpy as jnp
from jax.experimental import pallas as pl


def kernel(x, edge_index, edge_weight, W_z, b_z, W_r, b_r, W_h, b_h, W_lin, b_lin):
    raise NotImplementedError("write your pallas kernel here")


# speedup vs baseline: 1.5615x; 1.5615x over previous
import functools

import jax
import jax.numpy as jnp
from jax.experimental import pallas as pl
from jax.experimental.pallas import tpu as pltpu

F_IN = 128
FILTERS = 128
BLOCK = 2048


def _fused_gru_head(x_ref, w_ref, b_ref, wlin_ref, blin_ref, out_ref):
    w = w_ref[0] + w_ref[1]
    y = jnp.dot(x_ref[...], w, preferred_element_type=jnp.float32)
    y = y + b_ref[...]
    z = jax.nn.sigmoid(y[:, :FILTERS])
    h_tilde = jnp.tanh(y[:, FILTERS:])
    h = jnp.maximum((1.0 - z) * h_tilde, 0.0)
    out_ref[...] = (
        jnp.sum(h * wlin_ref[...], axis=1, keepdims=True) + blin_ref[0, 0]
    )


@functools.partial(jax.jit, static_argnames=())
def kernel(x, edge_index, edge_weight, W_z, b_z, W_r, b_r, W_h, b_h,
           W_lin, b_lin):
    del edge_index, edge_weight, W_r, b_r
    n = x.shape[0]

    w_stack = jnp.concatenate(
        [W_z[:, 0, :F_IN, :], W_h[:, 0, :F_IN, :]], axis=2
    )
    b_cat = jnp.concatenate([b_z, b_h]).reshape(1, 2 * FILTERS)
    wlin = W_lin.reshape(1, FILTERS)
    blin = b_lin.reshape(1, 1)

    grid = (pl.cdiv(n, BLOCK),)
    out = pl.pallas_call(
        _fused_gru_head,
        grid=grid,
        in_specs=[
            pl.BlockSpec((BLOCK, F_IN), lambda i: (i, 0)),
            pl.BlockSpec((2, F_IN, 2 * FILTERS), lambda i: (0, 0, 0)),
            pl.BlockSpec((1, 2 * FILTERS), lambda i: (0, 0)),
            pl.BlockSpec((1, FILTERS), lambda i: (0, 0)),
            pl.BlockSpec((1, 1), lambda i: (0, 0)),
        ],
        out_specs=pl.BlockSpec((BLOCK, 1), lambda i: (i, 0)),
        out_shape=jax.ShapeDtypeStruct((n, 1), x.dtype),
        compiler_params=pltpu.CompilerParams(
            dimension_semantics=("arbitrary",),
        ),
    )(x, w_stack, b_cat, wlin, blin)
    return out
